# Initial kernel scaffold; baseline (speedup 1.0000x reference)
import jax
import jax.numpy as jnp
from jax.experimental import pallas as pl


def kernel(hidden_states, w1, w2, topk_weights, topk_ids):
    raise NotImplementedError("write your pallas kernel here")


# speedup vs baseline: 1.9038x; 1.9038x over previous
import functools

import jax
import jax.numpy as jnp
from jax import lax
from jax.experimental import pallas as pl
from jax.experimental.pallas import tpu as pltpu

NUM_EXPERTS = 8
TOP_K = 2
D_MODEL = 768
D_FF = 768
M_TOKENS = 2048

BM = 512


def _moe_dense_kernel(ids_ref, tw_ref, x_ref, w1_ref, w2_ref, out_ref):
    e = pl.program_id(1)
    x = x_ref[...]
    w1 = w1_ref[0]
    w2 = w2_ref[0]
    h = lax.dot_general(x, w1, (((1,), (1,)), ((), ())),
                        preferred_element_type=jnp.float32)
    gate = h[:, :D_FF]
    up = h[:, D_FF:]
    act = gate * jax.nn.sigmoid(gate) * up
    y = lax.dot_general(act, w2, (((1,), (1,)), ((), ())),
                        preferred_element_type=jnp.float32)
    ids = ids_ref[...]
    tw = tw_ref[...]
    cw = jnp.sum(jnp.where(ids == e, tw, 0.0), axis=1)
    contrib = y * cw[:, None]

    @pl.when(e == 0)
    def _():
        out_ref[...] = contrib

    @pl.when(e != 0)
    def _():
        out_ref[...] += contrib


def kernel(hidden_states, w1, w2, topk_weights, topk_ids):
    ids = topk_ids.astype(jnp.int32)
    grid = (M_TOKENS // BM, NUM_EXPERTS)
    return pl.pallas_call(
        _moe_dense_kernel,
        grid=grid,
        in_specs=[
            pl.BlockSpec((BM, TOP_K), lambda m, e: (m, 0)),
            pl.BlockSpec((BM, TOP_K), lambda m, e: (m, 0)),
            pl.BlockSpec((BM, D_MODEL), lambda m, e: (m, 0)),
            pl.BlockSpec((1, 2 * D_FF, D_MODEL), lambda m, e: (e, 0, 0)),
            pl.BlockSpec((1, D_MODEL, D_FF), lambda m, e: (e, 0, 0)),
        ],
        out_specs=pl.BlockSpec((BM, D_MODEL), lambda m, e: (m, 0)),
        out_shape=jax.ShapeDtypeStruct((M_TOKENS, D_MODEL), jnp.float32),
    )(ids, topk_weights, hidden_states, w1, w2)
